# Initial kernel scaffold; baseline (speedup 1.0000x reference)
import jax
import jax.numpy as jnp
from jax.experimental import pallas as pl


def kernel(x, table):
    raise NotImplementedError("write your pallas kernel here")


# speedup vs baseline: 3.9300x; 3.9300x over previous
import functools

import numpy as np
import jax
import jax.numpy as jnp
from jax import lax
from jax.experimental import pallas as pl
from jax.experimental.pallas import tpu as pltpu
from jax.experimental.pallas import tpu_sc as plsc

_VOCAB = 100000
_EMBED = 128
_WINDOW = 200
_BATCH = 1024
_SCALE = float(np.sqrt(float(_EMBED)))

_NC = 2
_NS = 16
_NW = _NC * _NS
_ROWS_PER_W = _BATCH // _NW
_HALF = _WINDOW // 2


def _positional_encoding(length, depth):
    pos = np.arange(length)[:, np.newaxis]
    i = np.arange(depth)[np.newaxis, :]
    val = pos / 10000 ** (2 * (i // 2) / depth)
    pe = np.concatenate([np.sin(val[:, 0::2]), np.cos(val[:, 1::2])], axis=-1)
    return pe.astype(np.float32)


_POS = _positional_encoding(_WINDOW, _EMBED)


def _sc_body(x_hbm, pos_hbm, table_hbm, out_hbm, idx_v, rows_v, pos_v, sem):
    wid = lax.axis_index("s") * _NC + lax.axis_index("c")
    pltpu.sync_copy(pos_hbm, pos_v)

    def row_body(i, carry):
        b = wid * _ROWS_PER_W + i
        pltpu.sync_copy(x_hbm.at[b], idx_v)
        cp0 = pltpu.async_copy(
            table_hbm.at[idx_v.at[0]], rows_v.at[pl.ds(0, _HALF)], sem)
        cp1 = pltpu.async_copy(
            table_hbm.at[idx_v.at[1]], rows_v.at[pl.ds(_HALF, _HALF)], sem)
        cp0.wait()
        cp1.wait()

        def tok_body(t, c2):
            for v in range(_EMBED // 16):
                sl = (t, pl.ds(v * 16, 16))
                rows_v[sl] = rows_v[sl] * _SCALE + pos_v[sl]
            return c2

        lax.fori_loop(0, _WINDOW, tok_body, 0)
        pltpu.sync_copy(rows_v, out_hbm.at[b])
        return carry

    lax.fori_loop(0, _ROWS_PER_W, row_body, 0)


@functools.partial(jax.jit, static_argnames=())
def kernel(x, table):
    x3 = x.reshape(_BATCH, 2, _HALF)
    pos = jnp.asarray(_POS)
    mesh = plsc.VectorSubcoreMesh(core_axis_name="c", subcore_axis_name="s")
    call = functools.partial(
        pl.kernel,
        mesh=mesh,
        out_type=jax.ShapeDtypeStruct((_BATCH, _WINDOW, _EMBED), jnp.float32),
        scratch_types=[
            pltpu.VMEM((2, _HALF), jnp.int32),
            pltpu.VMEM((_WINDOW, _EMBED), jnp.float32),
            pltpu.VMEM((_WINDOW, _EMBED), jnp.float32),
            pltpu.SemaphoreType.DMA,
        ],
    )(_sc_body)
    return call(x3, pos, table)


# speedup vs baseline: 6.3417x; 1.6137x over previous
import functools

import numpy as np
import jax
import jax.numpy as jnp
from jax import lax
from jax.experimental import pallas as pl
from jax.experimental.pallas import tpu as pltpu
from jax.experimental.pallas import tpu_sc as plsc

_VOCAB = 100000
_EMBED = 128
_WINDOW = 200
_BATCH = 1024
_SCALE = float(np.sqrt(float(_EMBED)))

_NC = 2
_NS = 16
_NW = _NC * _NS
_ROWS_PER_W = _BATCH // _NW
_HALF = _WINDOW // 2
_PAIRS = _ROWS_PER_W // 2


def _positional_encoding(length, depth):
    pos = np.arange(length)[:, np.newaxis]
    i = np.arange(depth)[np.newaxis, :]
    val = pos / 10000 ** (2 * (i // 2) / depth)
    pe = np.concatenate([np.sin(val[:, 0::2]), np.cos(val[:, 1::2])], axis=-1)
    return pe.astype(np.float32)


_POS = _positional_encoding(_WINDOW, _EMBED)


def _sc_body(x_hbm, pos_hbm, table_hbm, out_hbm,
             idx_v, rows0, rows1, pos_v, sg0, sg1, sw0, sw1):
    wid = lax.axis_index("s") * _NC + lax.axis_index("c")
    base = wid * _ROWS_PER_W
    pltpu.sync_copy(pos_hbm, pos_v)
    pltpu.sync_copy(x_hbm.at[wid], idx_v)

    def start_gather(r, buf, sem):
        pltpu.async_copy(table_hbm.at[idx_v.at[r, 0]],
                         buf.at[pl.ds(0, _HALF)], sem)
        pltpu.async_copy(table_hbm.at[idx_v.at[r, 1]],
                         buf.at[pl.ds(_HALF, _HALF)], sem)

    def wait_gather(buf, sem):
        pltpu.make_async_copy(table_hbm.at[pl.ds(0, _WINDOW)], buf, sem).wait()

    def start_wb(buf, r, sem):
        pltpu.async_copy(buf, out_hbm.at[base + r], sem)

    def wait_wb(buf, sem):
        pltpu.make_async_copy(buf, out_hbm.at[0], sem).wait()

    def compute(buf):
        def tok(t, c):
            for u in range(2):
                tt = t * 2 + u
                for v in range(_EMBED // 16):
                    sl = (tt, pl.ds(v * 16, 16))
                    buf[sl] = buf[sl] * _SCALE + pos_v[sl]
            return c
        lax.fori_loop(0, _WINDOW // 2, tok, 0)

    start_gather(0, rows0, sg0)

    def pair(j, carry):
        @pl.when(j > 0)
        def _():
            wait_wb(rows1, sw1)
        start_gather(2 * j + 1, rows1, sg1)
        wait_gather(rows0, sg0)
        compute(rows0)
        start_wb(rows0, 2 * j, sw0)

        @pl.when(j < _PAIRS - 1)
        def _():
            wait_wb(rows0, sw0)
            start_gather(2 * j + 2, rows0, sg0)
        wait_gather(rows1, sg1)
        compute(rows1)
        start_wb(rows1, 2 * j + 1, sw1)
        return carry

    lax.fori_loop(0, _PAIRS, pair, 0)
    wait_wb(rows0, sw0)
    wait_wb(rows1, sw1)


@jax.jit
def kernel(x, table):
    x4 = x.reshape(_NW, _ROWS_PER_W, 2, _HALF)
    pos = jnp.asarray(_POS)
    mesh = plsc.VectorSubcoreMesh(core_axis_name="c", subcore_axis_name="s")
    call = functools.partial(
        pl.kernel,
        mesh=mesh,
        out_type=jax.ShapeDtypeStruct((_BATCH, _WINDOW, _EMBED), jnp.float32),
        scratch_types=[
            pltpu.VMEM((_ROWS_PER_W, 2, _HALF), jnp.int32),
            pltpu.VMEM((_WINDOW, _EMBED), jnp.float32),
            pltpu.VMEM((_WINDOW, _EMBED), jnp.float32),
            pltpu.VMEM((_WINDOW, _EMBED), jnp.float32),
            pltpu.SemaphoreType.DMA,
            pltpu.SemaphoreType.DMA,
            pltpu.SemaphoreType.DMA,
            pltpu.SemaphoreType.DMA,
        ],
    )(_sc_body)
    return call(x4, pos, table)
